# baseline (speedup 1.0000x reference)
import functools

import jax
import jax.numpy as jnp
from jax import lax
from jax.experimental import pallas as pl
from jax.experimental.pallas import tpu as pltpu
from jax.experimental.pallas import tpu_sc as plsc

B, T, D = 4096, 200, 64
NW = 32
ROWS_PER_W = B // NW
LANES = 16
C_TILES = D // LANES


def _body(x_hbm, tok_hbm, pos_hbm, out_hbm, idx_v, pos_v, buf_v, gsem):
    wid = lax.axis_index("s") * 2 + lax.axis_index("c")
    base = wid * ROWS_PER_W

    pltpu.sync_copy(x_hbm.at[pl.ds(base, ROWS_PER_W)], idx_v)
    pltpu.sync_copy(pos_hbm, pos_v)

    def chunk(step, carry):
        b = base + step
        d1 = pltpu.make_async_copy(
            tok_hbm.at[idx_v.at[step, pl.ds(0, 128)]],
            buf_v.at[pl.ds(0, 128)], gsem)
        d2 = pltpu.make_async_copy(
            tok_hbm.at[idx_v.at[step, pl.ds(128, 72)]],
            buf_v.at[pl.ds(128, 72)], gsem)
        d1.start()
        d2.start()
        d1.wait()
        d2.wait()

        def add_row(r, c2):
            for c in range(C_TILES):
                sl = pl.ds(c * LANES, LANES)
                buf_v[r, sl] = buf_v[r, sl] + pos_v[r, sl]
            return c2

        lax.fori_loop(0, T, add_row, 0)

        pltpu.sync_copy(buf_v, out_hbm.at[b])
        return carry

    lax.fori_loop(0, ROWS_PER_W, chunk, 0)


@jax.jit
def kernel(x, token_table, pos_table):
    mesh = plsc.VectorSubcoreMesh(core_axis_name="c", subcore_axis_name="s")
    k = functools.partial(
        pl.kernel,
        out_type=jax.ShapeDtypeStruct((B, T, D), jnp.float32),
        mesh=mesh,
        scratch_types=[
            pltpu.VMEM((ROWS_PER_W, T), jnp.int32),
            pltpu.VMEM((T, D), jnp.float32),
            pltpu.VMEM((T, D), jnp.float32),
            pltpu.SemaphoreType.DMA,
        ],
        compiler_params=pltpu.CompilerParams(use_tc_tiling_on_sc=False),
    )(_body)
    return k(x, token_table, pos_table)


# speedup vs baseline: 1.1221x; 1.1221x over previous
import functools

import jax
import jax.numpy as jnp
from jax import lax
from jax.experimental import pallas as pl
from jax.experimental.pallas import tpu as pltpu
from jax.experimental.pallas import tpu_sc as plsc

B, T, D = 4096, 200, 64
NW = 32
ROWS_PER_W = B // NW


def _body(x_hbm, tok_hbm, pos_hbm, out_hbm, idx_v, spos, buf0, buf1,
          gsem, isem, wsem0, wsem1):
    sid = lax.axis_index("s")
    wid = sid * 2 + lax.axis_index("c")
    base = wid * ROWS_PER_W

    pltpu.sync_copy(x_hbm.at[pl.ds(base, ROWS_PER_W)], idx_v)

    @pl.when(sid == 0)
    def _():
        pltpu.sync_copy(pos_hbm, spos)

    plsc.subcore_barrier()

    pltpu.sync_copy(spos, buf0)

    def gather_add(step, buf):
        d1 = pltpu.async_copy(
            tok_hbm.at[idx_v.at[step, pl.ds(0, 128)]],
            buf.at[pl.ds(0, 128)], gsem, add=True)
        d2 = pltpu.async_copy(
            tok_hbm.at[idx_v.at[step, pl.ds(128, 72)]],
            buf.at[pl.ds(128, 72)], gsem, add=True)
        return d1, d2

    def chunk(g, carry):
        p = lax.rem(g, 2)

        def run(buf, obuf, wsem, owsem):
            d1, d2 = gather_add(g, buf)

            @pl.when(jnp.logical_and(g >= 1, g + 1 < ROWS_PER_W))
            def _():
                pltpu.make_async_copy(obuf, out_hbm.at[base + g - 1], owsem).wait()

            @pl.when(g + 1 < ROWS_PER_W)
            def _():
                pltpu.async_copy(spos, obuf, isem)

            d1.wait()
            d2.wait()
            pltpu.async_copy(buf, out_hbm.at[base + g], wsem)

            @pl.when(g + 1 < ROWS_PER_W)
            def _():
                pltpu.make_async_copy(spos, obuf, isem).wait()

        @pl.when(p == 0)
        def _():
            run(buf0, buf1, wsem0, wsem1)

        @pl.when(p == 1)
        def _():
            run(buf1, buf0, wsem1, wsem0)

        return carry

    lax.fori_loop(0, ROWS_PER_W, chunk, 0)

    last = base + ROWS_PER_W - 1
    pltpu.make_async_copy(buf0, out_hbm.at[last], wsem0).wait()
    pltpu.make_async_copy(buf1, out_hbm.at[last], wsem1).wait()


@jax.jit
def kernel(x, token_table, pos_table):
    mesh = plsc.VectorSubcoreMesh(core_axis_name="c", subcore_axis_name="s")
    k = functools.partial(
        pl.kernel,
        out_type=jax.ShapeDtypeStruct((B, T, D), jnp.float32),
        mesh=mesh,
        scratch_types=[
            pltpu.VMEM((ROWS_PER_W, T), jnp.int32),
            pltpu.VMEM_SHARED((T, D), jnp.float32),
            pltpu.VMEM((T, D), jnp.float32),
            pltpu.VMEM((T, D), jnp.float32),
            pltpu.SemaphoreType.DMA,
            pltpu.SemaphoreType.DMA,
            pltpu.SemaphoreType.DMA,
            pltpu.SemaphoreType.DMA,
        ],
        compiler_params=pltpu.CompilerParams(use_tc_tiling_on_sc=False),
    )(_body)
    return k(x, token_table, pos_table)
